# baseline (speedup 1.0000x reference)
import functools

import jax
import jax.numpy as jnp
from jax import lax
from jax.experimental import pallas as pl
from jax.experimental.pallas import tpu as pltpu
from jax.experimental.pallas import tpu_sc as plsc

N_NODES = 10000
NPAD = 10240
NE = 170000
NTILES = 16
GRP = 128
NGRP = 84
EPT = NGRP * GRP
EP = NTILES * EPT


def _mm(x, w, bn=400, bp=512, bk=256):
    m, k = x.shape
    _, p = w.shape
    nk = k // bk
    assert m % bn == 0 and p % bp == 0 and k % bk == 0

    def body(x_ref, w_ref, o_ref, acc_ref):
        @pl.when(pl.program_id(2) == 0)
        def _init():
            acc_ref[...] = jnp.zeros_like(acc_ref)

        acc_ref[...] += jnp.dot(x_ref[...], w_ref[...],
                                preferred_element_type=jnp.float32)

        @pl.when(pl.program_id(2) == nk - 1)
        def _fin():
            o_ref[...] = acc_ref[...]

    return pl.pallas_call(
        body,
        grid=(m // bn, p // bp, nk),
        in_specs=[pl.BlockSpec((bn, bk), lambda i, j, kk: (i, kk)),
                  pl.BlockSpec((bk, bp), lambda i, j, kk: (kk, j))],
        out_specs=pl.BlockSpec((bn, bp), lambda i, j, kk: (i, j)),
        scratch_shapes=[pltpu.VMEM((bn, bp), jnp.float32)],
        out_shape=jax.ShapeDtypeStruct((m, p), jnp.float32),
    )(x, w)


def _edge_softmax_body(a_sT, a_dT, src2, dst2, e_out, den_out,
                       asv, adv, srcv, dstv, ev, denv, tmp, acc, shared_den,
                       *, heads):
    cid = lax.axis_index("c")
    sid = lax.axis_index("s")
    hsc = heads // 2
    base = sid * EPT
    row0 = sid * 640
    lane = lax.iota(jnp.int32, 16)
    z16 = jnp.zeros((16,), jnp.float32)

    pltpu.sync_copy(src2.at[sid], srcv)
    pltpu.sync_copy(dst2.at[sid], dstv)

    for h_loc in range(hsc):
        h = cid * hsc + h_loc
        pltpu.sync_copy(a_sT.at[h], asv)
        pltpu.sync_copy(a_dT.at[h], adv)

        def zero_body(i, _):
            denv[pl.ds(pl.multiple_of(i * 16, 16), 16)] = z16
            return 0
        lax.fori_loop(0, NPAD // 16, zero_body, 0)

        def edge_body(i, _):
            sl = pl.ds(pl.multiple_of(i * 16, 16), 16)
            sv = srcv[sl]
            dv = dstv[sl]
            av = plsc.load_gather(asv, [sv]) + plsc.load_gather(adv, [dv])
            av = jnp.where(av >= 0.0, av, 0.2 * av)
            e16 = jnp.exp(av)
            gid = base + i * 16 + lane
            e16 = jnp.where(gid < NE, e16, 0.0)
            ev[sl] = e16
            plsc.addupdate_scatter(denv, [dv], e16)
            return 0
        lax.fori_loop(0, EPT // 16, edge_body, 0)

        pltpu.sync_copy(ev, e_out.at[h, sid])
        pltpu.sync_copy(denv, shared_den.at[sid])
        plsc.subcore_barrier()

        for t in range(NTILES):
            pltpu.sync_copy(shared_den.at[t, pl.ds(row0, 640)], tmp.at[t])

        def red_body(i, _):
            sl = pl.ds(pl.multiple_of(i * 16, 16), 16)
            s = tmp[0, sl]
            for t in range(1, NTILES):
                s = s + tmp[t, sl]
            acc[sl] = s
            return 0
        lax.fori_loop(0, 640 // 16, red_body, 0)

        pltpu.sync_copy(acc, den_out.at[h, pl.ds(row0, 640)])
        plsc.subcore_barrier()


_SC_PARAMS = pltpu.CompilerParams(needs_layout_passes=False)


def _edge_softmax(a_sT, a_dT, src2, dst2, heads):
    mesh = plsc.VectorSubcoreMesh(core_axis_name="c", subcore_axis_name="s")
    return pl.kernel(
        functools.partial(_edge_softmax_body, heads=heads),
        compiler_params=_SC_PARAMS,
        out_type=(jax.ShapeDtypeStruct((heads, NTILES, EPT), jnp.float32),
                  jax.ShapeDtypeStruct((heads, NPAD), jnp.float32)),
        mesh=mesh,
        scratch_types=[
            pltpu.VMEM((N_NODES,), jnp.float32),
            pltpu.VMEM((N_NODES,), jnp.float32),
            pltpu.VMEM((EPT,), jnp.int32),
            pltpu.VMEM((EPT,), jnp.int32),
            pltpu.VMEM((EPT,), jnp.float32),
            pltpu.VMEM((NPAD,), jnp.float32),
            pltpu.VMEM((NTILES, 640), jnp.float32),
            pltpu.VMEM((640,), jnp.float32),
            pltpu.VMEM_SHARED((NTILES, NPAD), jnp.float32),
        ],
    )(a_sT, a_dT, src2, dst2)


GRP2 = 64
NG2 = EPT // GRP2
DROW = NG2 // 2
HALF_E = EPT // 2
HALF_G = HALF_E // GRP2


def _agg_body(hflat, e3, src2, dst3, zhbm, out,
              idxv, dstv2, evg0, evg1, rbuf0, rbuf1, sbuf0, sbuf1,
              gsem0, gsem1, ssem0, ssem1, esem0, esem1, accum,
              *, nchunks, heads):
    dch = nchunks // heads
    cid = lax.axis_index("c")
    sid = lax.axis_index("s")
    row0 = sid * 640

    pltpu.sync_copy(dst3.at[sid], dstv2)

    def loff(gl):
        return pl.ds(pl.multiple_of(gl * GRP2, GRP2), GRP2)

    def didx(gg):
        return dstv2.at[gg // 2, pl.ds(pl.multiple_of((gg % 2) * GRP2, GRP2),
                                       GRP2)]

    def start_g(gl, buf, sem):
        pltpu.async_copy(hflat.at[idxv.at[loff(gl)]], buf, sem)

    def wait_g(gl, buf, sem):
        pltpu.make_async_copy(hflat.at[idxv.at[loff(gl)]], buf, sem).wait()

    def start_s(gg, buf, sem):
        pltpu.async_copy(buf, accum.at[didx(gg)], sem, add=True)

    def wait_s(gg, buf, sem):
        pltpu.make_async_copy(buf, accum.at[didx(gg)], sem).wait()

    def start_e(h, gg, buf, sem):
        pltpu.async_copy(e3.at[h, sid, loff(gg)], buf, sem)

    def wait_e(h, gg, buf, sem):
        pltpu.make_async_copy(e3.at[h, sid, loff(gg)], buf, sem).wait()

    def scale(src_buf, ebuf, dst_buf):
        def sc_body(j, _):
            r0 = 2 * j
            r1 = r0 + 1
            ea = plsc.load_gather(ebuf, [jnp.full((16,), r0, jnp.int32)])
            eb = plsc.load_gather(ebuf, [jnp.full((16,), r1, jnp.int32)])
            for t in range(8):
                sl = pl.ds(t * 16, 16)
                dst_buf[r0, sl] = src_buf[r0, sl] * ea
                dst_buf[r1, sl] = src_buf[r1, sl] * eb
            return 0
        lax.fori_loop(0, GRP2 // 2, sc_body, 0)

    def chunk_body(kc, _):
        c = cid + 2 * kc
        h = c // dch

        @pl.when(sid < NTILES - 1)
        def _z_full():
            pltpu.sync_copy(zhbm.at[pl.ds(0, 640)],
                            accum.at[pl.ds(row0, 640)])

        @pl.when(sid == NTILES - 1)
        def _z_tail():
            pltpu.sync_copy(zhbm.at[pl.ds(0, 400)],
                            accum.at[pl.ds(row0, 400)])

        first_barrier = [True]

        for half in range(2):
            g_base = half * HALF_G
            pltpu.sync_copy(
                src2.at[sid, pl.ds(half * HALF_E, HALF_E)], idxv)

            def idx_body(i, _):
                sl = pl.ds(pl.multiple_of(i * 16, 16), 16)
                idxv[sl] = idxv[sl] * nchunks + c
                return 0
            lax.fori_loop(0, HALF_E // 16, idx_body, 0)

            if first_barrier[0]:
                plsc.subcore_barrier()
                first_barrier[0] = False

            start_e(h, g_base + 0, evg0, esem0)
            start_g(0, rbuf0, gsem0)
            start_e(h, g_base + 1, evg1, esem1)
            start_g(1, rbuf1, gsem1)

            def pair(i, _):
                gl0 = 2 * i
                gl1 = gl0 + 1
                gg0 = g_base + gl0
                gg1 = g_base + gl1

                wait_g(gl0, rbuf0, gsem0)
                wait_e(h, gg0, evg0, esem0)

                @pl.when(jnp.logical_or(i > 0, half > 0))
                def _ws0():
                    wait_s(gg0 - 2, sbuf0, ssem0)

                scale(rbuf0, evg0, sbuf0)

                @pl.when(gl0 + 2 < HALF_G)
                def _ng0():
                    start_g(gl0 + 2, rbuf0, gsem0)
                    start_e(h, gg0 + 2, evg0, esem0)

                start_s(gg0, sbuf0, ssem0)

                wait_g(gl1, rbuf1, gsem1)
                wait_e(h, gg1, evg1, esem1)

                @pl.when(jnp.logical_or(i > 0, half > 0))
                def _ws1():
                    wait_s(gg1 - 2, sbuf1, ssem1)

                scale(rbuf1, evg1, sbuf1)

                @pl.when(gl1 + 2 < HALF_G)
                def _ng1():
                    start_g(gl1 + 2, rbuf1, gsem1)
                    start_e(h, gg1 + 2, evg1, esem1)

                start_s(gg1, sbuf1, ssem1)
                return 0
            lax.fori_loop(0, HALF_G // 2, pair, 0)

        wait_s(NG2 - 2, sbuf0, ssem0)
        wait_s(NG2 - 1, sbuf1, ssem1)

        plsc.subcore_barrier()

        @pl.when(sid < NTILES - 1)
        def _d_full():
            pltpu.sync_copy(accum.at[pl.ds(row0, 640)],
                            out.at[pl.ds(c * N_NODES + row0, 640)])

        @pl.when(sid == NTILES - 1)
        def _d_tail():
            pltpu.sync_copy(accum.at[pl.ds(row0, 400)],
                            out.at[pl.ds(c * N_NODES + row0, 400)])

        plsc.subcore_barrier()
        return 0

    lax.fori_loop(0, nchunks // 2, chunk_body, 0)


def _aggregate(hflat, e3, src2, dst3, zhbm, nchunks, heads):
    mesh = plsc.VectorSubcoreMesh(core_axis_name="c", subcore_axis_name="s")
    return pl.kernel(
        functools.partial(_agg_body, nchunks=nchunks, heads=heads),
        compiler_params=_SC_PARAMS,
        out_type=jax.ShapeDtypeStruct((nchunks * N_NODES, 128), jnp.float32),
        mesh=mesh,
        scratch_types=[
            pltpu.VMEM((HALF_E,), jnp.int32),
            pltpu.VMEM((DROW, 128), jnp.int32),
            pltpu.VMEM((GRP2,), jnp.float32),
            pltpu.VMEM((GRP2,), jnp.float32),
            pltpu.VMEM((GRP2, 128), jnp.float32),
            pltpu.VMEM((GRP2, 128), jnp.float32),
            pltpu.VMEM((GRP2, 128), jnp.float32),
            pltpu.VMEM((GRP2, 128), jnp.float32),
            pltpu.SemaphoreType.DMA,
            pltpu.SemaphoreType.DMA,
            pltpu.SemaphoreType.DMA,
            pltpu.SemaphoreType.DMA,
            pltpu.SemaphoreType.DMA,
            pltpu.SemaphoreType.DMA,
            pltpu.VMEM_SHARED((N_NODES, 128), jnp.float32),
        ],
    )(hflat, e3, src2, dst3, zhbm)


def _epilogue(outr, den, hres, bias, heads, nchunks, elu):
    n, p = hres.shape
    bn = 400
    dch = nchunks // heads

    def body(o_ref, d_ref, r_ref, b_ref, out_ref):
        inv = 1.0 / (d_ref[...] + 1e-16)
        parts = []
        for c in range(nchunks):
            parts.append(o_ref[c] * inv[:, c // dch:c // dch + 1])
        val = jnp.concatenate(parts, axis=1)
        if not elu:
            acc = val[:, :p]
            for hh in range(1, heads):
                acc = acc + val[:, hh * p:(hh + 1) * p]
            val = acc * (1.0 / heads)
        val = val + b_ref[...] + r_ref[...]
        if elu:
            val = jnp.where(val > 0.0, val, jnp.exp(val) - 1.0)
        out_ref[...] = val

    return pl.pallas_call(
        body,
        grid=(n // bn,),
        in_specs=[
            pl.BlockSpec((nchunks, bn, 128), lambda i: (0, i, 0)),
            pl.BlockSpec((bn, heads), lambda i: (i, 0)),
            pl.BlockSpec((bn, p), lambda i: (i, 0)),
            pl.BlockSpec((1, p), lambda i: (0, 0)),
        ],
        out_specs=pl.BlockSpec((bn, p), lambda i: (i, 0)),
        out_shape=jax.ShapeDtypeStruct((n, p), jnp.float32),
    )(outr, den, hres, bias)


def _fold_att(W, a_src, a_dst, heads):
    c = W.shape[0]
    d = a_src.shape[1]
    Wr = W.reshape(c, heads, d)
    ws = jnp.einsum("chd,hd->ch", Wr, a_src)
    wd = jnp.einsum("chd,hd->ch", Wr, a_dst)
    return jnp.concatenate(
        [ws, wd, jnp.zeros((c, 128 - 2 * heads), jnp.float32)], axis=1)


def kernel(x, edge_index, original_size, W1, a_src1, a_dst1, b1, Wl1, bl1,
           W2, a_src2, a_dst2, b2, Wl2, bl2, W3, a_src3, a_dst3, b3, Wl3,
           bl3):
    n = x.shape[0]
    loop = jnp.arange(n, dtype=jnp.int32)
    src = jnp.concatenate([edge_index[0].astype(jnp.int32), loop])
    dst = jnp.concatenate([edge_index[1].astype(jnp.int32), loop])
    srcp = jnp.pad(src, (0, EP - NE))
    dstp = jnp.pad(dst, (0, EP - NE))
    src2 = srcp.reshape(NTILES, EPT)
    dst2 = dstp.reshape(NTILES, EPT)
    dst3 = dstp.reshape(NTILES, DROW, 128)
    zhbm = jnp.zeros((640, 128), jnp.float32)

    def gat_layer(h_in, W, Wsd, Wl, bias_total, heads, nchunks, elu):
        p_out = Wl.shape[1]
        hmat = _mm(h_in, W)
        hres = _mm(h_in, Wl, bp=min(512, p_out))
        asd = _mm(h_in, Wsd, bp=128)
        a_sT = asd[:, :heads].T
        a_dT = asd[:, heads:2 * heads].T
        e3, den = _edge_softmax(a_sT, a_dT, src2, dst2, heads)
        den_n = den[:, :n].T
        hflat = hmat.reshape(n * nchunks, 128)
        outr = _aggregate(hflat, e3, src2, dst3, zhbm, nchunks, heads)
        outr3 = outr.reshape(nchunks, n, 128)
        return _epilogue(outr3, den_n, hres, bias_total.reshape(1, p_out),
                         heads, nchunks, elu)

    h1 = gat_layer(x, W1, _fold_att(W1, a_src1, a_dst1, 4), Wl1,
                   b1 + bl1, 4, 8, True)
    h2 = gat_layer(h1, W2, _fold_att(W2, a_src2, a_dst2, 4), Wl2,
                   b2 + bl2, 4, 8, True)
    out = gat_layer(h2, W3, _fold_att(W3, a_src3, a_dst3, 6), Wl3,
                    b3 + bl3, 6, 12, False)
    return out


# speedup vs baseline: 1.6047x; 1.6047x over previous
import functools

import jax
import jax.numpy as jnp
from jax import lax
from jax.experimental import pallas as pl
from jax.experimental.pallas import tpu as pltpu
from jax.experimental.pallas import tpu_sc as plsc

N_NODES = 10000
NPAD = 10240
NE = 170000
NTILES = 16
GRP = 128
NGRP = 84
EPT = NGRP * GRP
EP = NTILES * EPT


def _mm(x, w, bn=400, bp=512, bk=256):
    m, k = x.shape
    _, p = w.shape
    nk = k // bk
    assert m % bn == 0 and p % bp == 0 and k % bk == 0

    def body(x_ref, w_ref, o_ref, acc_ref):
        @pl.when(pl.program_id(2) == 0)
        def _init():
            acc_ref[...] = jnp.zeros_like(acc_ref)

        acc_ref[...] += jnp.dot(x_ref[...], w_ref[...],
                                preferred_element_type=jnp.float32)

        @pl.when(pl.program_id(2) == nk - 1)
        def _fin():
            o_ref[...] = acc_ref[...]

    return pl.pallas_call(
        body,
        grid=(m // bn, p // bp, nk),
        in_specs=[pl.BlockSpec((bn, bk), lambda i, j, kk: (i, kk)),
                  pl.BlockSpec((bk, bp), lambda i, j, kk: (kk, j))],
        out_specs=pl.BlockSpec((bn, bp), lambda i, j, kk: (i, j)),
        scratch_shapes=[pltpu.VMEM((bn, bp), jnp.float32)],
        out_shape=jax.ShapeDtypeStruct((m, p), jnp.float32),
    )(x, w)


def _edge_softmax_body(a_sT, a_dT, src2, dst2, e_out, den_out,
                       asv, adv, srcv, dstv, ev, denv, tmp, acc, shared_den,
                       *, heads):
    cid = lax.axis_index("c")
    sid = lax.axis_index("s")
    hsc = heads // 2
    base = sid * EPT
    row0 = sid * 640
    lane = lax.iota(jnp.int32, 16)
    z16 = jnp.zeros((16,), jnp.float32)

    pltpu.sync_copy(src2.at[sid], srcv)
    pltpu.sync_copy(dst2.at[sid], dstv)

    for h_loc in range(hsc):
        h = cid * hsc + h_loc
        pltpu.sync_copy(a_sT.at[h], asv)
        pltpu.sync_copy(a_dT.at[h], adv)

        def zero_body(i, _):
            denv[pl.ds(pl.multiple_of(i * 16, 16), 16)] = z16
            return 0
        lax.fori_loop(0, NPAD // 16, zero_body, 0)

        def edge_body(i, _):
            sl = pl.ds(pl.multiple_of(i * 16, 16), 16)
            sv = srcv[sl]
            dv = dstv[sl]
            av = plsc.load_gather(asv, [sv]) + plsc.load_gather(adv, [dv])
            av = jnp.where(av >= 0.0, av, 0.2 * av)
            e16 = jnp.exp(av)
            gid = base + i * 16 + lane
            e16 = jnp.where(gid < NE, e16, 0.0)
            ev[sl] = e16
            plsc.addupdate_scatter(denv, [dv], e16)
            return 0
        lax.fori_loop(0, EPT // 16, edge_body, 0)

        pltpu.sync_copy(ev, e_out.at[h, sid])
        pltpu.sync_copy(denv, shared_den.at[sid])
        plsc.subcore_barrier()

        for t in range(NTILES):
            pltpu.sync_copy(shared_den.at[t, pl.ds(row0, 640)], tmp.at[t])

        def red_body(i, _):
            sl = pl.ds(pl.multiple_of(i * 16, 16), 16)
            s = tmp[0, sl]
            for t in range(1, NTILES):
                s = s + tmp[t, sl]
            acc[sl] = s
            return 0
        lax.fori_loop(0, 640 // 16, red_body, 0)

        pltpu.sync_copy(acc, den_out.at[h, pl.ds(row0, 640)])
        plsc.subcore_barrier()


_SC_PARAMS = pltpu.CompilerParams(needs_layout_passes=False)


def _edge_softmax(a_sT, a_dT, src2, dst2, heads):
    mesh = plsc.VectorSubcoreMesh(core_axis_name="c", subcore_axis_name="s")
    return pl.kernel(
        functools.partial(_edge_softmax_body, heads=heads),
        compiler_params=_SC_PARAMS,
        out_type=(jax.ShapeDtypeStruct((heads, NTILES, EPT), jnp.float32),
                  jax.ShapeDtypeStruct((heads, NPAD), jnp.float32)),
        mesh=mesh,
        scratch_types=[
            pltpu.VMEM((N_NODES,), jnp.float32),
            pltpu.VMEM((N_NODES,), jnp.float32),
            pltpu.VMEM((EPT,), jnp.int32),
            pltpu.VMEM((EPT,), jnp.int32),
            pltpu.VMEM((EPT,), jnp.float32),
            pltpu.VMEM((NPAD,), jnp.float32),
            pltpu.VMEM((NTILES, 640), jnp.float32),
            pltpu.VMEM((640,), jnp.float32),
            pltpu.VMEM_SHARED((NTILES, NPAD), jnp.float32),
        ],
    )(a_sT, a_dT, src2, dst2)


GRP2 = 128
NG2 = EPT // GRP2
HALF_G = NG2 // 2
HALF_E = EPT // 2


def _agg_body(hflat, e3, src2, dst3, zhbm, out,
              idxv, dstv2, evg0, evg1, rbuf0, rbuf1,
              gsem0, gsem1, ssem0, ssem1, esem0, esem1, accum,
              *, nchunks, heads):
    dch = nchunks // heads
    cid = lax.axis_index("c")
    sid = lax.axis_index("s")
    row0 = sid * 640

    pltpu.sync_copy(dst3.at[sid], dstv2)

    def loff(gl):
        return pl.ds(pl.multiple_of(gl * GRP2, GRP2), GRP2)

    def start_g(gl, buf, sem):
        pltpu.async_copy(hflat.at[idxv.at[loff(gl)]], buf, sem)

    def wait_g(gl, buf, sem):
        pltpu.make_async_copy(hflat.at[idxv.at[loff(gl)]], buf, sem).wait()

    def start_s(gg, buf, sem):
        pltpu.async_copy(buf, accum.at[dstv2.at[gg]], sem, add=True)

    def wait_s(gg, buf, sem):
        pltpu.make_async_copy(buf, accum.at[dstv2.at[gg]], sem).wait()

    def start_e(h, gg, buf, sem):
        pltpu.async_copy(e3.at[h, sid, loff(gg)], buf, sem)

    def wait_e(h, gg, buf, sem):
        pltpu.make_async_copy(e3.at[h, sid, loff(gg)], buf, sem).wait()

    def scale(buf, ebuf):
        def sc_body(j, _):
            r0 = 2 * j
            r1 = r0 + 1
            ea = plsc.load_gather(ebuf, [jnp.full((16,), r0, jnp.int32)])
            eb = plsc.load_gather(ebuf, [jnp.full((16,), r1, jnp.int32)])
            for t in range(8):
                sl = pl.ds(t * 16, 16)
                buf[r0, sl] = buf[r0, sl] * ea
                buf[r1, sl] = buf[r1, sl] * eb
            return 0
        lax.fori_loop(0, GRP2 // 2, sc_body, 0)

    def chunk_body(kc, _):
        c = cid + 2 * kc
        h = c // dch

        @pl.when(sid < NTILES - 1)
        def _z_full():
            pltpu.sync_copy(zhbm.at[pl.ds(0, 640)],
                            accum.at[pl.ds(row0, 640)])

        @pl.when(sid == NTILES - 1)
        def _z_tail():
            pltpu.sync_copy(zhbm.at[pl.ds(0, 400)],
                            accum.at[pl.ds(row0, 400)])

        first_barrier = [True]

        for half in range(2):
            g_base = half * HALF_G
            pltpu.sync_copy(
                src2.at[sid, pl.ds(half * HALF_E, HALF_E)], idxv)

            def idx_body(i, _):
                sl = pl.ds(pl.multiple_of(i * 16, 16), 16)
                idxv[sl] = idxv[sl] * nchunks + c
                return 0
            lax.fori_loop(0, HALF_E // 16, idx_body, 0)

            if first_barrier[0]:
                plsc.subcore_barrier()
                first_barrier[0] = False

            start_e(h, g_base + 0, evg0, esem0)
            start_g(0, rbuf0, gsem0)
            start_e(h, g_base + 1, evg1, esem1)
            start_g(1, rbuf1, gsem1)

            def pair(i, _):
                gl0 = 2 * i
                gl1 = gl0 + 1
                gg0 = g_base + gl0
                gg1 = g_base + gl1

                wait_g(gl0, rbuf0, gsem0)
                wait_e(h, gg0, evg0, esem0)
                scale(rbuf0, evg0)
                start_s(gg0, rbuf0, ssem0)

                wait_g(gl1, rbuf1, gsem1)
                wait_e(h, gg1, evg1, esem1)
                scale(rbuf1, evg1)
                start_s(gg1, rbuf1, ssem1)

                @pl.when(i < HALF_G // 2 - 1)
                def _nxt():
                    wait_s(gg0, rbuf0, ssem0)
                    start_g(gl0 + 2, rbuf0, gsem0)
                    start_e(h, gg0 + 2, evg0, esem0)
                    wait_s(gg1, rbuf1, ssem1)
                    start_g(gl1 + 2, rbuf1, gsem1)
                    start_e(h, gg1 + 2, evg1, esem1)
                return 0
            lax.fori_loop(0, HALF_G // 2, pair, 0)

            wait_s(g_base + HALF_G - 2, rbuf0, ssem0)
            wait_s(g_base + HALF_G - 1, rbuf1, ssem1)

        plsc.subcore_barrier()

        @pl.when(sid < NTILES - 1)
        def _d_full():
            pltpu.sync_copy(accum.at[pl.ds(row0, 640)],
                            out.at[pl.ds(c * N_NODES + row0, 640)])

        @pl.when(sid == NTILES - 1)
        def _d_tail():
            pltpu.sync_copy(accum.at[pl.ds(row0, 400)],
                            out.at[pl.ds(c * N_NODES + row0, 400)])

        plsc.subcore_barrier()
        return 0

    lax.fori_loop(0, nchunks // 2, chunk_body, 0)


def _aggregate(hflat, e3, src2, dst3, zhbm, nchunks, heads):
    mesh = plsc.VectorSubcoreMesh(core_axis_name="c", subcore_axis_name="s")
    return pl.kernel(
        functools.partial(_agg_body, nchunks=nchunks, heads=heads),
        compiler_params=_SC_PARAMS,
        out_type=jax.ShapeDtypeStruct((nchunks * N_NODES, 128), jnp.float32),
        mesh=mesh,
        scratch_types=[
            pltpu.VMEM((HALF_E,), jnp.int32),
            pltpu.VMEM((NG2, GRP2), jnp.int32),
            pltpu.VMEM((GRP2,), jnp.float32),
            pltpu.VMEM((GRP2,), jnp.float32),
            pltpu.VMEM((GRP2, 128), jnp.float32),
            pltpu.VMEM((GRP2, 128), jnp.float32),
            pltpu.SemaphoreType.DMA,
            pltpu.SemaphoreType.DMA,
            pltpu.SemaphoreType.DMA,
            pltpu.SemaphoreType.DMA,
            pltpu.SemaphoreType.DMA,
            pltpu.SemaphoreType.DMA,
            pltpu.VMEM_SHARED((N_NODES, 128), jnp.float32),
        ],
    )(hflat, e3, src2, dst3, zhbm)


def _epilogue(outr, den, hres, bias, heads, nchunks, elu):
    n, p = hres.shape
    bn = 400
    dch = nchunks // heads

    def body(o_ref, d_ref, r_ref, b_ref, out_ref):
        inv = 1.0 / (d_ref[...] + 1e-16)
        parts = []
        for c in range(nchunks):
            parts.append(o_ref[c] * inv[:, c // dch:c // dch + 1])
        val = jnp.concatenate(parts, axis=1)
        if not elu:
            acc = val[:, :p]
            for hh in range(1, heads):
                acc = acc + val[:, hh * p:(hh + 1) * p]
            val = acc * (1.0 / heads)
        val = val + b_ref[...] + r_ref[...]
        if elu:
            val = jnp.where(val > 0.0, val, jnp.exp(val) - 1.0)
        out_ref[...] = val

    return pl.pallas_call(
        body,
        grid=(n // bn,),
        in_specs=[
            pl.BlockSpec((nchunks, bn, 128), lambda i: (0, i, 0)),
            pl.BlockSpec((bn, heads), lambda i: (i, 0)),
            pl.BlockSpec((bn, p), lambda i: (i, 0)),
            pl.BlockSpec((1, p), lambda i: (0, 0)),
        ],
        out_specs=pl.BlockSpec((bn, p), lambda i: (i, 0)),
        out_shape=jax.ShapeDtypeStruct((n, p), jnp.float32),
    )(outr, den, hres, bias)


def _fold_att(W, a_src, a_dst, heads):
    c = W.shape[0]
    d = a_src.shape[1]
    Wr = W.reshape(c, heads, d)
    ws = jnp.einsum("chd,hd->ch", Wr, a_src)
    wd = jnp.einsum("chd,hd->ch", Wr, a_dst)
    return jnp.concatenate(
        [ws, wd, jnp.zeros((c, 128 - 2 * heads), jnp.float32)], axis=1)


def kernel(x, edge_index, original_size, W1, a_src1, a_dst1, b1, Wl1, bl1,
           W2, a_src2, a_dst2, b2, Wl2, bl2, W3, a_src3, a_dst3, b3, Wl3,
           bl3):
    n = x.shape[0]
    loop = jnp.arange(n, dtype=jnp.int32)
    src = jnp.concatenate([edge_index[0].astype(jnp.int32), loop])
    dst = jnp.concatenate([edge_index[1].astype(jnp.int32), loop])
    srcp = jnp.pad(src, (0, EP - NE))
    dstp = jnp.pad(dst, (0, EP - NE))
    src2 = srcp.reshape(NTILES, EPT)
    dst2 = dstp.reshape(NTILES, EPT)
    dst3 = dstp.reshape(NTILES, NG2, GRP2)
    zhbm = jnp.zeros((640, 128), jnp.float32)

    def gat_layer(h_in, W, Wsd, Wl, bias_total, heads, nchunks, elu):
        p_out = Wl.shape[1]
        hmat = _mm(h_in, W)
        hres = _mm(h_in, Wl, bp=min(512, p_out))
        asd = _mm(h_in, Wsd, bp=128)
        a_sT = asd[:, :heads].T
        a_dT = asd[:, heads:2 * heads].T
        e3, den = _edge_softmax(a_sT, a_dT, src2, dst2, heads)
        den_n = den[:, :n].T
        hflat = hmat.reshape(n * nchunks, 128)
        outr = _aggregate(hflat, e3, src2, dst3, zhbm, nchunks, heads)
        outr3 = outr.reshape(nchunks, n, 128)
        return _epilogue(outr3, den_n, hres, bias_total.reshape(1, p_out),
                         heads, nchunks, elu)

    h1 = gat_layer(x, W1, _fold_att(W1, a_src1, a_dst1, 4), Wl1,
                   b1 + bl1, 4, 8, True)
    h2 = gat_layer(h1, W2, _fold_att(W2, a_src2, a_dst2, 4), Wl2,
                   b2 + bl2, 4, 8, True)
    out = gat_layer(h2, W3, _fold_att(W3, a_src3, a_dst3, 6), Wl3,
                    b3 + bl3, 6, 12, False)
    return out
